# Initial kernel scaffold; baseline (speedup 1.0000x reference)
import jax
import jax.numpy as jnp
from jax.experimental import pallas as pl


def kernel(x, emb, W_ih0, W_hh0, b_ih0, b_hh0, W_ih1, W_hh1, b_ih1, b_hh1, fc_W, fc_b):
    raise NotImplementedError("write your pallas kernel here")


# speedup vs baseline: 3.1878x; 3.1878x over previous
import jax
import jax.numpy as jnp
from jax.experimental import pallas as pl
from jax.experimental.pallas import tpu as pltpu
from jax.experimental.pallas import tpu_sc as plsc


def _sc_gather(emb, idx_flat, window=128):
    n_tok = idx_flat.shape[0]
    full_e = emb.shape[1]
    split = full_e // 128
    emb = emb.reshape(-1, 128)
    idx_flat = (
        idx_flat[:, None] * split
        + jnp.arange(split, dtype=jnp.int32)[None, :]
    ).reshape(-1)
    n = idx_flat.shape[0]
    e_dim = 128
    idx2 = idx_flat.reshape(1, n)
    mesh = plsc.VectorSubcoreMesh(core_axis_name="core", subcore_axis_name="subcore")

    @pl.kernel(out_type=jax.ShapeDtypeStruct((n, e_dim), emb.dtype), mesh=mesh)
    def gather_kernel(emb_hbm, i_hbm, o_hbm):
        def body(i_vmem, o_vmem):
            pltpu.sync_copy(emb_hbm.at[i_vmem.at[0]], o_vmem)

        pltpu.emit_pipeline(
            body,
            grid=(n // window,),
            in_specs=[pl.BlockSpec((1, window), lambda i: (0, i))],
            out_specs=[pl.BlockSpec((window, e_dim), lambda i: (i, 0))],
            core_axis_name=("core", "subcore"),
            dimension_semantics=(pltpu.PARALLEL,),
        )(i_hbm, o_hbm)

    return gather_kernel(emb, idx2).reshape(n_tok, full_e)


def _in_gates(lhs_bf, w_t_bf, bias2):
    n = lhs_bf.shape[0]
    f = w_t_bf.shape[1]

    def body(l_ref, w_ref, b_ref, o_ref):
        o_ref[...] = (
            jnp.dot(l_ref[...], w_ref[...], preferred_element_type=jnp.float32)
            + b_ref[...]
        )

    return pl.pallas_call(
        body,
        out_shape=jax.ShapeDtypeStruct((n, f), jnp.float32),
    )(lhs_bf, w_t_bf, bias2)


def _lstm_rec(xg, whh_t_bf):
    s, bn, f4 = xg.shape
    h_dim = f4 // 4

    def body(xg_ref, w_ref, hseq_ref, h_ref, c_ref, hs, cs):
        t = pl.program_id(0)

        @pl.when(t == 0)
        def _():
            hs[...] = jnp.zeros_like(hs)
            cs[...] = jnp.zeros_like(cs)

        g = xg_ref[0] + jnp.dot(
            hs[...].astype(jnp.bfloat16), w_ref[...],
            preferred_element_type=jnp.float32,
        )
        gi = jax.nn.sigmoid(g[:, :h_dim])
        gf = jax.nn.sigmoid(g[:, h_dim:2 * h_dim])
        gg = jnp.tanh(g[:, 2 * h_dim:3 * h_dim])
        go = jax.nn.sigmoid(g[:, 3 * h_dim:])
        c = gf * cs[...] + gi * gg
        h = go * jnp.tanh(c)
        cs[...] = c
        hs[...] = h
        hseq_ref[0] = h

        @pl.when(t == s - 1)
        def _():
            h_ref[...] = h
            c_ref[...] = c

    return pl.pallas_call(
        body,
        grid=(s,),
        in_specs=[
            pl.BlockSpec((1, bn, f4), lambda t: (t, 0, 0)),
            pl.BlockSpec((h_dim, f4), lambda t: (0, 0)),
        ],
        out_specs=[
            pl.BlockSpec((1, bn, h_dim), lambda t: (t, 0, 0)),
            pl.BlockSpec((bn, h_dim), lambda t: (0, 0)),
            pl.BlockSpec((bn, h_dim), lambda t: (0, 0)),
        ],
        out_shape=[
            jax.ShapeDtypeStruct((s, bn, h_dim), jnp.float32),
            jax.ShapeDtypeStruct((bn, h_dim), jnp.float32),
            jax.ShapeDtypeStruct((bn, h_dim), jnp.float32),
        ],
        scratch_shapes=[
            pltpu.VMEM((bn, h_dim), jnp.float32),
            pltpu.VMEM((bn, h_dim), jnp.float32),
        ],
    )(xg, whh_t_bf)


def _fc(lhs_bf, w_t_bf, bias2, vt=3200):
    n, h_dim = lhs_bf.shape
    v = w_t_bf.shape[1]

    def body(l_ref, w_ref, b_ref, o_ref):
        o_ref[...] = (
            jnp.dot(l_ref[...], w_ref[...], preferred_element_type=jnp.float32)
            + b_ref[...]
        )

    return pl.pallas_call(
        body,
        grid=(v // vt,),
        in_specs=[
            pl.BlockSpec((n, h_dim), lambda j: (0, 0)),
            pl.BlockSpec((h_dim, vt), lambda j: (0, j)),
            pl.BlockSpec((1, vt), lambda j: (0, j)),
        ],
        out_specs=pl.BlockSpec((n, vt), lambda j: (0, j)),
        out_shape=jax.ShapeDtypeStruct((n, v), jnp.float32),
    )(lhs_bf, w_t_bf, bias2)


def kernel(x, emb, W_ih0, W_hh0, b_ih0, b_hh0, W_ih1, W_hh1, b_ih1, b_hh1, fc_W, fc_b):
    bn, s = x.shape
    v, e_dim = emb.shape
    h_dim = W_hh0.shape[1]
    n = bn * s
    bf = jnp.bfloat16

    idx_flat = x.T.reshape(-1).astype(jnp.int32)
    e_sb = _sc_gather(emb, idx_flat)

    xg0 = _in_gates(
        e_sb.astype(bf), W_ih0.astype(bf).T, (b_ih0 + b_hh0).reshape(1, -1)
    ).reshape(s, bn, 4 * h_dim)
    h0seq, h0, c0 = _lstm_rec(xg0, W_hh0.astype(bf).T)

    xg1 = _in_gates(
        h0seq.reshape(n, h_dim).astype(bf),
        W_ih1.astype(bf).T,
        (b_ih1 + b_hh1).reshape(1, -1),
    ).reshape(s, bn, 4 * h_dim)
    h1seq, h1, c1 = _lstm_rec(xg1, W_hh1.astype(bf).T)

    o1_bs = jnp.swapaxes(h1seq, 0, 1).reshape(n, h_dim)
    logits = _fc(
        o1_bs.astype(bf), fc_W.astype(bf).T, fc_b.reshape(1, -1)
    ).reshape(bn, s, v)

    h_out = jnp.stack([h0, h1], axis=0)
    c_out = jnp.stack([c0, c1], axis=0)
    return (logits, h_out, c_out)


# speedup vs baseline: 3.7085x; 1.1634x over previous
import jax
import jax.numpy as jnp
from jax.experimental import pallas as pl
from jax.experimental.pallas import tpu as pltpu
from jax.experimental.pallas import tpu_sc as plsc


def _sc_gather(emb, idx_flat, window=128):
    n_tok = idx_flat.shape[0]
    full_e = emb.shape[1]
    split = full_e // 128
    emb = emb.reshape(-1, 128)
    idx_flat = (
        idx_flat[:, None] * split
        + jnp.arange(split, dtype=jnp.int32)[None, :]
    ).reshape(-1)
    n = idx_flat.shape[0]
    e_dim = 128
    idx2 = idx_flat.reshape(1, n)
    mesh = plsc.VectorSubcoreMesh(core_axis_name="core", subcore_axis_name="subcore")

    @pl.kernel(out_type=jax.ShapeDtypeStruct((n, e_dim), emb.dtype), mesh=mesh)
    def gather_kernel(emb_hbm, i_hbm, o_hbm):
        def body(i_vmem, o_vmem):
            pltpu.sync_copy(emb_hbm.at[i_vmem.at[0]], o_vmem)

        pltpu.emit_pipeline(
            body,
            grid=(n // window,),
            in_specs=[pl.BlockSpec((1, window), lambda i: (0, i))],
            out_specs=[pl.BlockSpec((window, e_dim), lambda i: (i, 0))],
            core_axis_name=("core", "subcore"),
            dimension_semantics=(pltpu.PARALLEL,),
        )(i_hbm, o_hbm)

    return gather_kernel(emb, idx2).reshape(n_tok, full_e)


_DN_T = (((1,), (1,)), ((), ()))


def _in_gates(lhs_bf, w_raw, bias2):
    n = lhs_bf.shape[0]
    f = w_raw.shape[0]

    def body(l_ref, w_ref, b_ref, o_ref):
        o_ref[...] = (
            jax.lax.dot_general(
                l_ref[...], w_ref[...].astype(jnp.bfloat16), _DN_T,
                preferred_element_type=jnp.float32,
            )
            + b_ref[...]
        )

    return pl.pallas_call(
        body,
        out_shape=jax.ShapeDtypeStruct((n, f), jnp.float32),
    )(lhs_bf, w_raw, bias2)


def _lstm_rec(xg, whh_t_bf):
    s, bn, f4 = xg.shape
    h_dim = f4 // 4

    def body(xg_ref, w_ref, hseq_ref, h_ref, c_ref, hs, cs):
        t = pl.program_id(0)

        @pl.when(t == 0)
        def _():
            hs[...] = jnp.zeros_like(hs)
            cs[...] = jnp.zeros_like(cs)

        g = xg_ref[0] + jnp.dot(
            hs[...].astype(jnp.bfloat16), w_ref[...],
            preferred_element_type=jnp.float32,
        )
        gi = jax.nn.sigmoid(g[:, :h_dim])
        gf = jax.nn.sigmoid(g[:, h_dim:2 * h_dim])
        gg = jnp.tanh(g[:, 2 * h_dim:3 * h_dim])
        go = jax.nn.sigmoid(g[:, 3 * h_dim:])
        c = gf * cs[...] + gi * gg
        h = go * jnp.tanh(c)
        cs[...] = c
        hs[...] = h
        hseq_ref[0] = h

        @pl.when(t == s - 1)
        def _():
            h_ref[...] = h
            c_ref[...] = c

    return pl.pallas_call(
        body,
        grid=(s,),
        in_specs=[
            pl.BlockSpec((1, bn, f4), lambda t: (t, 0, 0)),
            pl.BlockSpec((h_dim, f4), lambda t: (0, 0)),
        ],
        out_specs=[
            pl.BlockSpec((1, bn, h_dim), lambda t: (t, 0, 0)),
            pl.BlockSpec((bn, h_dim), lambda t: (0, 0)),
            pl.BlockSpec((bn, h_dim), lambda t: (0, 0)),
        ],
        out_shape=[
            jax.ShapeDtypeStruct((s, bn, h_dim), jnp.float32),
            jax.ShapeDtypeStruct((bn, h_dim), jnp.float32),
            jax.ShapeDtypeStruct((bn, h_dim), jnp.float32),
        ],
        scratch_shapes=[
            pltpu.VMEM((bn, h_dim), jnp.float32),
            pltpu.VMEM((bn, h_dim), jnp.float32),
        ],
    )(xg, whh_t_bf)


def _fc(lhs_bf, w_raw, bias2, vt=1280):
    n, h_dim = lhs_bf.shape
    v = w_raw.shape[0]

    def body(l_ref, w_ref, b_ref, o_ref):
        o_ref[...] = (
            jax.lax.dot_general(
                l_ref[...], w_ref[...].astype(jnp.bfloat16), _DN_T,
                preferred_element_type=jnp.float32,
            )
            + b_ref[...]
        )

    return pl.pallas_call(
        body,
        grid=(v // vt,),
        in_specs=[
            pl.BlockSpec((n, h_dim), lambda j: (0, 0)),
            pl.BlockSpec((vt, h_dim), lambda j: (j, 0)),
            pl.BlockSpec((1, vt), lambda j: (0, j)),
        ],
        out_specs=pl.BlockSpec((n, vt), lambda j: (0, j)),
        out_shape=jax.ShapeDtypeStruct((n, v), jnp.float32),
    )(lhs_bf, w_raw, bias2)


def kernel(x, emb, W_ih0, W_hh0, b_ih0, b_hh0, W_ih1, W_hh1, b_ih1, b_hh1, fc_W, fc_b):
    bn, s = x.shape
    v, e_dim = emb.shape
    h_dim = W_hh0.shape[1]
    n = bn * s
    bf = jnp.bfloat16

    idx_flat = x.T.reshape(-1).astype(jnp.int32)
    e_sb = _sc_gather(emb, idx_flat)

    xg0 = _in_gates(
        e_sb.astype(bf), W_ih0, (b_ih0 + b_hh0).reshape(1, -1)
    ).reshape(s, bn, 4 * h_dim)
    h0seq, h0, c0 = _lstm_rec(xg0, W_hh0.astype(bf).T)

    xg1 = _in_gates(
        h0seq.reshape(n, h_dim).astype(bf),
        W_ih1,
        (b_ih1 + b_hh1).reshape(1, -1),
    ).reshape(s, bn, 4 * h_dim)
    h1seq, h1, c1 = _lstm_rec(xg1, W_hh1.astype(bf).T)

    o1_bs = jnp.swapaxes(h1seq, 0, 1).reshape(n, h_dim)
    logits = _fc(
        o1_bs.astype(bf), fc_W, fc_b.reshape(1, -1)
    ).reshape(bn, s, v)

    h_out = jnp.stack([h0, h1], axis=0)
    c_out = jnp.stack([c0, c1], axis=0)
    return (logits, h_out, c_out)
